# baseline (speedup 1.0000x reference)
import functools
import math

import jax
import jax.numpy as jnp
from jax import lax
from jax.experimental import pallas as pl
from jax.experimental.pallas import tpu as pltpu
from jax.experimental.pallas import tpu_sc as plsc

VOCAB = 1000000
D = 128
BATCH = 4096
SEQ = 200

NC = 2
NS = 16
NW = NC * NS
B = BATCH * SEQ
B_PER_W = B // NW
CHUNK = 128
NCHUNK = B_PER_W // CHUNK
LANES = 16
SCALE = math.sqrt(D)


def _emb_body(table_hbm, idx_hbm, out_hbm,
              idx_v, gbuf0, gbuf1, sbuf0, sbuf1,
              gsem0, gsem1, ssem0, ssem1):
    wid = lax.axis_index("s") * NC + lax.axis_index("c")
    base = wid * B_PER_W

    pltpu.sync_copy(idx_hbm.at[wid], idx_v)

    pltpu.async_copy(table_hbm.at[idx_v.at[0]], gbuf0, gsem0)
    pltpu.async_copy(table_hbm.at[idx_v.at[1]], gbuf1, gsem1)

    def scale_chunk(src, dst):
        def row(r, _):
            for c in range(D // LANES):
                sl = pl.ds(c * LANES, LANES)
                dst[r, sl] = src[r, sl] * SCALE
            return 0
        lax.fori_loop(0, CHUNK, row, 0, unroll=4)

    def step(g, _):
        for slot, (gbuf, sbuf, gsem, ssem) in enumerate(
                ((gbuf0, sbuf0, gsem0, ssem0), (gbuf1, sbuf1, gsem1, ssem1))):
            j = 2 * g + slot
            pltpu.make_async_copy(table_hbm.at[idx_v.at[j]], gbuf, gsem).wait()

            @pl.when(g > 0)
            def _():
                pltpu.make_async_copy(
                    sbuf, out_hbm.at[pl.ds(base + (j - 2) * CHUNK, CHUNK)],
                    ssem).wait()

            scale_chunk(gbuf, sbuf)
            pltpu.async_copy(
                sbuf, out_hbm.at[pl.ds(base + j * CHUNK, CHUNK)], ssem)

            @pl.when(g < NCHUNK // 2 - 1)
            def _():
                pltpu.async_copy(table_hbm.at[idx_v.at[j + 2]], gbuf, gsem)
        return 0

    lax.fori_loop(0, NCHUNK // 2, step, 0)

    pltpu.make_async_copy(
        sbuf0, out_hbm.at[pl.ds(base + (NCHUNK - 2) * CHUNK, CHUNK)],
        ssem0).wait()
    pltpu.make_async_copy(
        sbuf1, out_hbm.at[pl.ds(base + (NCHUNK - 1) * CHUNK, CHUNK)],
        ssem1).wait()


@jax.jit
def kernel(x, table):
    mesh = plsc.VectorSubcoreMesh(core_axis_name="c", subcore_axis_name="s")
    fn = pl.kernel(
        _emb_body,
        out_type=jax.ShapeDtypeStruct((B, D), jnp.float32),
        mesh=mesh,
        scratch_types=[
            pltpu.VMEM((NCHUNK, CHUNK), jnp.int32),
            pltpu.VMEM((CHUNK, D), jnp.float32),
            pltpu.VMEM((CHUNK, D), jnp.float32),
            pltpu.VMEM((CHUNK, D), jnp.float32),
            pltpu.VMEM((CHUNK, D), jnp.float32),
            pltpu.SemaphoreType.DMA,
            pltpu.SemaphoreType.DMA,
            pltpu.SemaphoreType.DMA,
            pltpu.SemaphoreType.DMA,
        ],
        name="sc_embedding_lookup",
    )
    idx = x.reshape(NW, NCHUNK, CHUNK)
    out = fn(table, idx)
    return out.reshape(BATCH, SEQ, D)
